# baseline (speedup 1.0000x reference)
import functools

import jax
import jax.numpy as jnp
from jax import lax
from jax.experimental import pallas as pl
from jax.experimental.pallas import tpu as pltpu
from jax.experimental.pallas import tpu_sc as plsc

N = 50000
F_IN = 1433
HID = 16
NCLS = 7

NC = 2
NS = 16
NW = NC * NS
LANES = 128
CH = 16
T = 400
E2 = NW * T * LANES
G = T // CH

NP = 51200
RP = NP // 8
ROWS_PER_TILE = NP // NS
NDUMP = 1024

_mesh = plsc.VectorSubcoreMesh(core_axis_name="c", subcore_axis_name="s")


@functools.partial(
    pl.kernel,
    mesh=_mesh,
    compiler_params=pltpu.CompilerParams(use_tc_tiling_on_sc=False),
    out_type=jax.ShapeDtypeStruct((NC * NP,), jnp.float32),
    scratch_types=[
        pltpu.VMEM_SHARED((NP,), jnp.float32),
        pltpu.VMEM((CH, LANES), jnp.int32),
        pltpu.VMEM((LANES,), jnp.float32),
        pltpu.VMEM((ROWS_PER_TILE,), jnp.float32),
    ],
)
def _deg_kernel(dst3, out, acc, dstb, ones_v, zbuf):
    cid = lax.axis_index("c")
    sid = lax.axis_index("s")
    wid = sid * NC + cid
    for i in range(LANES // 16):
        ones_v[pl.ds(i * 16, 16)] = jnp.ones((16,), jnp.float32)

    def zero(i, c):
        zbuf[pl.ds(i * 16, 16)] = jnp.zeros((16,), jnp.float32)
        return c

    lax.fori_loop(0, ROWS_PER_TILE // 16, zero, 0)
    pltpu.sync_copy(zbuf, acc.at[pl.ds(sid * ROWS_PER_TILE, ROWS_PER_TILE)])
    plsc.subcore_barrier()

    def body(g, c):
        pltpu.sync_copy(dst3.at[wid, pl.ds(g * CH, CH)], dstb)
        for b in range(CH):
            pltpu.sync_copy(ones_v, acc.at[dstb.at[b]], add=True)
        return c

    lax.fori_loop(0, G, body, 0)
    plsc.subcore_barrier()
    off = sid * ROWS_PER_TILE
    pltpu.sync_copy(acc.at[pl.ds(off, ROWS_PER_TILE)],
                    out.at[pl.ds(cid * NP + off, ROWS_PER_TILE)])


@functools.partial(
    pl.kernel,
    mesh=_mesh,
    compiler_params=pltpu.CompilerParams(use_tc_tiling_on_sc=False),
    out_type=jax.ShapeDtypeStruct((NC, NP, HID), jnp.float32),
    scratch_types=[
        pltpu.VMEM_SHARED((NP, HID), jnp.float32),
        pltpu.VMEM((2, CH, LANES), jnp.int32),
        pltpu.VMEM((2, CH, LANES), jnp.int32),
        pltpu.VMEM((2, CH, LANES, HID), jnp.float32),
        pltpu.SemaphoreType.DMA,
        pltpu.SemaphoreType.DMA,
    ],
)
def _mp_kernel(hs, src4, dst3, out, acc, srcb, dstb, rows, sem, sem2):
    cid = lax.axis_index("c")
    sid = lax.axis_index("s")
    wid = sid * NC + cid
    off = sid * ROWS_PER_TILE
    pltpu.sync_copy(hs.at[pl.ds(cid * NP + off, ROWS_PER_TILE)],
                    acc.at[pl.ds(off, ROWS_PER_TILE)])
    plsc.subcore_barrier()

    pltpu.sync_copy(src4.at[cid, wid, pl.ds(0, CH)], srcb.at[0])
    pltpu.sync_copy(dst3.at[wid, pl.ds(0, CH)], dstb.at[0])
    for b in range(CH):
        pltpu.async_copy(hs.at[srcb.at[0, b]], rows.at[0, b], sem)

    def body(g, c):
        p = lax.rem(g, 2)
        q = 1 - p

        for b in range(CH):
            pltpu.make_async_copy(hs.at[pl.ds(0, LANES)],
                                  rows.at[p, b], sem).wait()

        @pl.when(g > 0)
        def _drain_sc():
            for b in range(CH):
                pltpu.make_async_copy(rows.at[q, b],
                                      acc.at[pl.ds(0, LANES)], sem2).wait()

        @pl.when(g < G - 1)
        def _stage_fire():
            pltpu.sync_copy(src4.at[cid, wid, pl.ds((g + 1) * CH, CH)],
                            srcb.at[q])
            pltpu.sync_copy(dst3.at[wid, pl.ds((g + 1) * CH, CH)],
                            dstb.at[q])
            for b in range(CH):
                pltpu.async_copy(hs.at[srcb.at[q, b]], rows.at[q, b], sem)

        for b in range(CH):
            pltpu.async_copy(rows.at[p, b], acc.at[dstb.at[p, b]], sem2,
                             add=True)
        return c

    lax.fori_loop(0, G, body, 0)
    lastp = (G - 1) % 2
    for b in range(CH):
        pltpu.make_async_copy(rows.at[lastp, b],
                              acc.at[pl.ds(0, LANES)], sem2).wait()
    plsc.subcore_barrier()
    pltpu.sync_copy(acc.at[pl.ds(off, ROWS_PER_TILE)],
                    out.at[cid, pl.ds(off, ROWS_PER_TILE)])


BMN = 512
_GRID1 = (N + BMN - 1) // BMN
BR = 640
_GRIDP = RP // BR


def _dinvp_body(d0_ref, d1_ref, m_ref, o_ref):
    dinv = lax.rsqrt(d0_ref[...] + d1_ref[...] + 1.0)
    o_ref[...] = jnp.dot(dinv, m_ref[...], preferred_element_type=jnp.float32)


def _dinvp(d0, d1, m):
    return pl.pallas_call(
        _dinvp_body,
        grid=(_GRIDP,),
        in_specs=[
            pl.BlockSpec((BR, 8), lambda i: (i, 0)),
            pl.BlockSpec((BR, 8), lambda i: (i, 0)),
            pl.BlockSpec((8, LANES), lambda i: (0, 0)),
        ],
        out_specs=pl.BlockSpec((BR, LANES), lambda i: (i, 0)),
        out_shape=jax.ShapeDtypeStruct((RP, LANES), jnp.float32),
    )(d0, d1, m)


def _tc1_body(x_ref, w_ref, o_ref):
    o_ref[...] = lax.dot_general(x_ref[...], w_ref[...],
                                 (((0,), (0,)), ((), ())),
                                 preferred_element_type=jnp.float32)


def _tc1(xT, W1):
    return pl.pallas_call(
        _tc1_body,
        grid=(_GRID1,),
        in_specs=[
            pl.BlockSpec((F_IN, BMN), lambda i: (0, i)),
            pl.BlockSpec((F_IN, HID), lambda i: (0, 0)),
        ],
        out_specs=pl.BlockSpec((BMN, HID), lambda i: (i, 0)),
        out_shape=jax.ShapeDtypeStruct((NP, HID), jnp.float32),
    )(xT, W1)


def _mul_body(a_ref, b_ref, o_ref):
    v = a_ref[...] * b_ref[...]
    o_ref[0] = v
    o_ref[1] = v


def _mul(a, b):
    return pl.pallas_call(
        _mul_body,
        grid=(_GRIDP,),
        in_specs=[
            pl.BlockSpec((BR, LANES), lambda i: (i, 0)),
            pl.BlockSpec((BR, LANES), lambda i: (i, 0)),
        ],
        out_specs=pl.BlockSpec((2, BR, LANES), lambda i: (0, i, 0)),
        out_shape=jax.ShapeDtypeStruct((2, RP, LANES), jnp.float32),
    )(a, b)


def _mid_body(p_ref, hs_ref, dv_ref, b_ref, w_ref, o_ref):
    dv = dv_ref[...]
    agg = p_ref[0] + p_ref[1] - hs_ref[...]
    a = agg * dv + b_ref[...]
    o1 = jnp.where(a > 0, a, jnp.exp(a) - 1.0)
    v = jnp.dot(o1, w_ref[...], preferred_element_type=jnp.float32) * dv
    o_ref[0] = v
    o_ref[1] = v


def _mid(p, hs, dvp, b, Wbd):
    return pl.pallas_call(
        _mid_body,
        grid=(_GRIDP,),
        in_specs=[
            pl.BlockSpec((NC, BR, LANES), lambda i: (0, i, 0)),
            pl.BlockSpec((BR, LANES), lambda i: (i, 0)),
            pl.BlockSpec((BR, LANES), lambda i: (i, 0)),
            pl.BlockSpec((1, LANES), lambda i: (0, 0)),
            pl.BlockSpec((LANES, LANES), lambda i: (0, 0)),
        ],
        out_specs=pl.BlockSpec((2, BR, LANES), lambda i: (0, i, 0)),
        out_shape=jax.ShapeDtypeStruct((2, RP, LANES), jnp.float32),
    )(p, hs, dvp, b, Wbd)


def _fin_body(p_ref, hs_ref, dv_ref, b_ref, s_ref, o_ref):
    dv = dv_ref[...]
    agg = p_ref[0] + p_ref[1] - hs_ref[...]
    a = agg * dv + b_ref[...]
    z = jnp.where(a > 0, a, jnp.exp(a) - 1.0)
    col = lax.broadcasted_iota(jnp.int32, (BR, LANES), 1)
    valid = lax.rem(col, HID) < NCLS
    zm = jnp.where(valid, z, -jnp.inf)
    m = jnp.max(zm, axis=1, keepdims=True)
    e = jnp.where(valid, jnp.exp(z - m), 0.0)
    s = jnp.dot(e, s_ref[...], preferred_element_type=jnp.float32)
    o_ref[...] = z - m - jnp.log(s)


def _fin(p, hs, dvp, b, S):
    return pl.pallas_call(
        _fin_body,
        grid=(_GRIDP,),
        in_specs=[
            pl.BlockSpec((NC, BR, LANES), lambda i: (0, i, 0)),
            pl.BlockSpec((BR, LANES), lambda i: (i, 0)),
            pl.BlockSpec((BR, LANES), lambda i: (i, 0)),
            pl.BlockSpec((1, LANES), lambda i: (0, 0)),
            pl.BlockSpec((LANES, LANES), lambda i: (0, 0)),
        ],
        out_specs=pl.BlockSpec((BR, LANES), lambda i: (i, 0)),
        out_shape=jax.ShapeDtypeStruct((RP, LANES), jnp.float32),
    )(p, hs, dvp, b, S)


def kernel(x, edge_index, W1, b1, W2, b2, W3, b3):
    f32 = jnp.float32
    src = edge_index[0]
    dst = edge_index[1]
    pad = E2 - src.shape[0]
    src3 = jnp.concatenate(
        [src, jnp.zeros((pad,), jnp.int32)]).reshape(NW, T, LANES)
    src4 = jnp.stack([src3, src3 + NP])
    dump = N + jnp.arange(pad, dtype=jnp.int32) % NDUMP
    dst3 = jnp.concatenate([dst, dump]).reshape(NW, T, LANES)

    degp = _deg_kernel(dst3)
    d0 = degp[:NP].reshape(RP, 8)
    d1 = degp[NP:].reshape(RP, 8)
    eye8 = jnp.eye(8, dtype=f32)
    M = jnp.kron(eye8, jnp.ones((1, HID), f32))
    dvp = _dinvp(d0, d1, M)

    W2bd = jnp.kron(eye8, W2)
    W3bd = jnp.kron(eye8, jnp.pad(W3, ((0, 0), (0, HID - NCLS))))
    S16 = jnp.kron(eye8, jnp.ones((HID, HID), f32))
    b1t = jnp.tile(b1, 8).reshape(1, LANES)
    b2t = jnp.tile(b2, 8).reshape(1, LANES)
    b3t = jnp.tile(jnp.pad(b3, (0, HID - NCLS)), 8).reshape(1, LANES)

    hraw = _tc1(x.T, W1).reshape(RP, LANES)
    hs1 = _mul(hraw, dvp)
    p1 = _mp_kernel(hs1.reshape(2 * NP, HID), src4, dst3)
    hs2 = _mid(p1.reshape(NC, RP, LANES), hs1[0], dvp, b1t, W2bd)
    p2 = _mp_kernel(hs2.reshape(2 * NP, HID), src4, dst3)
    hs3 = _mid(p2.reshape(NC, RP, LANES), hs2[0], dvp, b2t, W3bd)
    p3 = _mp_kernel(hs3.reshape(2 * NP, HID), src4, dst3)
    outP = _fin(p3.reshape(NC, RP, LANES), hs3[0], dvp, b3t, S16)
    return outP.reshape(NP, HID)[:N, :NCLS]


# speedup vs baseline: 1.0435x; 1.0435x over previous
import functools

import jax
import jax.numpy as jnp
from jax import lax
from jax.experimental import pallas as pl
from jax.experimental.pallas import tpu as pltpu
from jax.experimental.pallas import tpu_sc as plsc

N = 50000
F_IN = 1433
HID = 16
NCLS = 7

NC = 2
NS = 16
NW = NC * NS
LANES = 128
CH = 16
T = 400
E2 = NW * T * LANES
G = T // CH

NP = 51200
RP = NP // 8
ROWS_PER_TILE = NP // NS
NDUMP = 1024

_mesh = plsc.VectorSubcoreMesh(core_axis_name="c", subcore_axis_name="s")


@functools.partial(
    pl.kernel,
    mesh=_mesh,
    compiler_params=pltpu.CompilerParams(use_tc_tiling_on_sc=False),
    out_type=jax.ShapeDtypeStruct((NC * NP,), jnp.float32),
    scratch_types=[
        pltpu.VMEM_SHARED((NP,), jnp.float32),
        pltpu.VMEM((CH, LANES), jnp.int32),
        pltpu.VMEM((LANES,), jnp.float32),
        pltpu.VMEM((ROWS_PER_TILE,), jnp.float32),
    ],
)
def _deg_kernel(dst3, out, acc, dstb, ones_v, zbuf):
    cid = lax.axis_index("c")
    sid = lax.axis_index("s")
    wid = sid * NC + cid
    for i in range(LANES // 16):
        ones_v[pl.ds(i * 16, 16)] = jnp.ones((16,), jnp.float32)

    def zero(i, c):
        zbuf[pl.ds(i * 16, 16)] = jnp.zeros((16,), jnp.float32)
        return c

    lax.fori_loop(0, ROWS_PER_TILE // 16, zero, 0)
    pltpu.sync_copy(zbuf, acc.at[pl.ds(sid * ROWS_PER_TILE, ROWS_PER_TILE)])
    plsc.subcore_barrier()

    def body(g, c):
        pltpu.sync_copy(dst3.at[wid, pl.ds(g * CH, CH)], dstb)
        for b in range(CH):
            pltpu.sync_copy(ones_v, acc.at[dstb.at[b]], add=True)
        return c

    lax.fori_loop(0, G, body, 0)
    plsc.subcore_barrier()
    off = sid * ROWS_PER_TILE
    pltpu.sync_copy(acc.at[pl.ds(off, ROWS_PER_TILE)],
                    out.at[pl.ds(cid * NP + off, ROWS_PER_TILE)])


@functools.partial(
    pl.kernel,
    mesh=_mesh,
    compiler_params=pltpu.CompilerParams(use_tc_tiling_on_sc=False),
    out_type=jax.ShapeDtypeStruct((NC, NP, HID), jnp.float32),
    scratch_types=[
        pltpu.VMEM_SHARED((NP, HID), jnp.float32),
        pltpu.VMEM((2, CH, LANES), jnp.int32),
        pltpu.VMEM((2, CH, LANES), jnp.int32),
        pltpu.VMEM((2, CH, LANES, HID), jnp.float32),
        pltpu.SemaphoreType.DMA,
        pltpu.SemaphoreType.DMA,
    ],
)
def _mp_kernel(hs, src3, dst3, out, acc, srcb, dstb, rows, sem, sem2):
    cid = lax.axis_index("c")
    sid = lax.axis_index("s")
    wid = sid * NC + cid
    off = sid * ROWS_PER_TILE
    pltpu.sync_copy(hs.at[pl.ds(off, ROWS_PER_TILE)],
                    acc.at[pl.ds(off, ROWS_PER_TILE)])
    plsc.subcore_barrier()

    def body(g, c):
        pltpu.sync_copy(src3.at[wid, pl.ds(g * CH, CH)], srcb.at[0])
        pltpu.sync_copy(dst3.at[wid, pl.ds(g * CH, CH)], dstb.at[0])
        for b in range(CH):
            pltpu.async_copy(hs.at[srcb.at[0, b]], rows.at[0, b], sem)
        for b in range(CH):
            pltpu.make_async_copy(hs.at[pl.ds(0, LANES)],
                                  rows.at[0, b], sem).wait()
        for b in range(CH):
            pltpu.sync_copy(rows.at[0, b], acc.at[dstb.at[0, b]], add=True)
        return c

    lax.fori_loop(0, G, body, 0)
    plsc.subcore_barrier()
    pltpu.sync_copy(acc.at[pl.ds(off, ROWS_PER_TILE)],
                    out.at[cid, pl.ds(off, ROWS_PER_TILE)])


BMN = 512
_GRID1 = (N + BMN - 1) // BMN
BR = 640
_GRIDP = RP // BR


def _dinvp_body(d0_ref, d1_ref, m_ref, o_ref):
    dinv = lax.rsqrt(d0_ref[...] + d1_ref[...] + 1.0)
    o_ref[...] = jnp.dot(dinv, m_ref[...], preferred_element_type=jnp.float32)


def _dinvp(d0, d1, m):
    return pl.pallas_call(
        _dinvp_body,
        grid=(_GRIDP,),
        in_specs=[
            pl.BlockSpec((BR, 8), lambda i: (i, 0)),
            pl.BlockSpec((BR, 8), lambda i: (i, 0)),
            pl.BlockSpec((8, LANES), lambda i: (0, 0)),
        ],
        out_specs=pl.BlockSpec((BR, LANES), lambda i: (i, 0)),
        out_shape=jax.ShapeDtypeStruct((RP, LANES), jnp.float32),
    )(d0, d1, m)


def _tc1_body(x_ref, w_ref, o_ref):
    o_ref[...] = lax.dot_general(x_ref[...], w_ref[...],
                                 (((0,), (0,)), ((), ())),
                                 preferred_element_type=jnp.float32)


def _tc1(xT, W1):
    return pl.pallas_call(
        _tc1_body,
        grid=(_GRID1,),
        in_specs=[
            pl.BlockSpec((F_IN, BMN), lambda i: (0, i)),
            pl.BlockSpec((F_IN, HID), lambda i: (0, 0)),
        ],
        out_specs=pl.BlockSpec((BMN, HID), lambda i: (i, 0)),
        out_shape=jax.ShapeDtypeStruct((NP, HID), jnp.float32),
    )(xT, W1)


def _mul_body(a_ref, b_ref, o_ref):
    o_ref[...] = a_ref[...] * b_ref[...]


def _mul(a, b):
    return pl.pallas_call(
        _mul_body,
        grid=(_GRIDP,),
        in_specs=[
            pl.BlockSpec((BR, LANES), lambda i: (i, 0)),
            pl.BlockSpec((BR, LANES), lambda i: (i, 0)),
        ],
        out_specs=pl.BlockSpec((BR, LANES), lambda i: (i, 0)),
        out_shape=jax.ShapeDtypeStruct((RP, LANES), jnp.float32),
    )(a, b)


def _mid_body(p_ref, hs_ref, dv_ref, b_ref, w_ref, o_ref):
    dv = dv_ref[...]
    agg = p_ref[0] + p_ref[1] - hs_ref[...]
    a = agg * dv + b_ref[...]
    o1 = jnp.where(a > 0, a, jnp.exp(a) - 1.0)
    o_ref[...] = jnp.dot(o1, w_ref[...],
                         preferred_element_type=jnp.float32) * dv


def _mid(p, hs, dvp, b, Wbd):
    return pl.pallas_call(
        _mid_body,
        grid=(_GRIDP,),
        in_specs=[
            pl.BlockSpec((NC, BR, LANES), lambda i: (0, i, 0)),
            pl.BlockSpec((BR, LANES), lambda i: (i, 0)),
            pl.BlockSpec((BR, LANES), lambda i: (i, 0)),
            pl.BlockSpec((1, LANES), lambda i: (0, 0)),
            pl.BlockSpec((LANES, LANES), lambda i: (0, 0)),
        ],
        out_specs=pl.BlockSpec((BR, LANES), lambda i: (i, 0)),
        out_shape=jax.ShapeDtypeStruct((RP, LANES), jnp.float32),
    )(p, hs, dvp, b, Wbd)


def _fin_body(p_ref, hs_ref, dv_ref, b_ref, s_ref, o_ref):
    dv = dv_ref[...]
    agg = p_ref[0] + p_ref[1] - hs_ref[...]
    a = agg * dv + b_ref[...]
    z = jnp.where(a > 0, a, jnp.exp(a) - 1.0)
    col = lax.broadcasted_iota(jnp.int32, (BR, LANES), 1)
    valid = lax.rem(col, HID) < NCLS
    zm = jnp.where(valid, z, -jnp.inf)
    m = jnp.max(zm, axis=1, keepdims=True)
    e = jnp.where(valid, jnp.exp(z - m), 0.0)
    s = jnp.dot(e, s_ref[...], preferred_element_type=jnp.float32)
    o_ref[...] = z - m - jnp.log(s)


def _fin(p, hs, dvp, b, S):
    return pl.pallas_call(
        _fin_body,
        grid=(_GRIDP,),
        in_specs=[
            pl.BlockSpec((NC, BR, LANES), lambda i: (0, i, 0)),
            pl.BlockSpec((BR, LANES), lambda i: (i, 0)),
            pl.BlockSpec((BR, LANES), lambda i: (i, 0)),
            pl.BlockSpec((1, LANES), lambda i: (0, 0)),
            pl.BlockSpec((LANES, LANES), lambda i: (0, 0)),
        ],
        out_specs=pl.BlockSpec((BR, LANES), lambda i: (i, 0)),
        out_shape=jax.ShapeDtypeStruct((RP, LANES), jnp.float32),
    )(p, hs, dvp, b, S)


def kernel(x, edge_index, W1, b1, W2, b2, W3, b3):
    f32 = jnp.float32
    src = edge_index[0]
    dst = edge_index[1]
    pad = E2 - src.shape[0]
    src3 = jnp.concatenate(
        [src, jnp.zeros((pad,), jnp.int32)]).reshape(NW, T, LANES)
    dump = N + jnp.arange(pad, dtype=jnp.int32) % NDUMP
    dst3 = jnp.concatenate([dst, dump]).reshape(NW, T, LANES)

    degp = _deg_kernel(dst3)
    d0 = degp[:NP].reshape(RP, 8)
    d1 = degp[NP:].reshape(RP, 8)
    eye8 = jnp.eye(8, dtype=f32)
    M = jnp.kron(eye8, jnp.ones((1, HID), f32))
    dvp = _dinvp(d0, d1, M)

    W2bd = jnp.kron(eye8, W2)
    W3bd = jnp.kron(eye8, jnp.pad(W3, ((0, 0), (0, HID - NCLS))))
    S16 = jnp.kron(eye8, jnp.ones((HID, HID), f32))
    b1t = jnp.tile(b1, 8).reshape(1, LANES)
    b2t = jnp.tile(b2, 8).reshape(1, LANES)
    b3t = jnp.tile(jnp.pad(b3, (0, HID - NCLS)), 8).reshape(1, LANES)

    hraw = _tc1(x.T, W1).reshape(RP, LANES)
    hs1 = _mul(hraw, dvp)
    p1 = _mp_kernel(hs1.reshape(NP, HID), src3, dst3)
    hs2 = _mid(p1.reshape(NC, RP, LANES), hs1, dvp, b1t, W2bd)
    p2 = _mp_kernel(hs2.reshape(NP, HID), src3, dst3)
    hs3 = _mid(p2.reshape(NC, RP, LANES), hs2, dvp, b2t, W3bd)
    p3 = _mp_kernel(hs3.reshape(NP, HID), src3, dst3)
    outP = _fin(p3.reshape(NC, RP, LANES), hs3, dvp, b3t, S16)
    return outP.reshape(NP, HID)[:N, :NCLS]


# speedup vs baseline: 1.4859x; 1.4240x over previous
import functools

import jax
import jax.numpy as jnp
from jax import lax
from jax.experimental import pallas as pl
from jax.experimental.pallas import tpu as pltpu
from jax.experimental.pallas import tpu_sc as plsc

N = 50000
F_IN = 1433
HID = 16
NCLS = 7

NC = 2
NS = 16
NW = NC * NS
LANES = 128
CH = 8
T = 392
E2 = NW * T * LANES
G = T // CH

NP = 50048
RP = NP // 8
ROWS_PER_TILE = NP // NS
ZB = 3136
NDUMP = 48

_mesh = plsc.VectorSubcoreMesh(core_axis_name="c", subcore_axis_name="s")


@functools.partial(
    pl.kernel,
    mesh=_mesh,
    compiler_params=pltpu.CompilerParams(use_tc_tiling_on_sc=False),
    out_type=jax.ShapeDtypeStruct((NC * NP,), jnp.float32),
    scratch_types=[
        pltpu.VMEM_SHARED((NP,), jnp.float32),
        pltpu.VMEM((CH, LANES), jnp.int32),
        pltpu.VMEM((LANES,), jnp.float32),
        pltpu.VMEM((ZB,), jnp.float32),
    ],
)
def _deg_kernel(dst3, out, acc, dstb, ones_v, zbuf):
    cid = lax.axis_index("c")
    sid = lax.axis_index("s")
    wid = sid * NC + cid
    for i in range(LANES // 16):
        ones_v[pl.ds(i * 16, 16)] = jnp.ones((16,), jnp.float32)

    def zero(i, c):
        zbuf[pl.ds(i * 16, 16)] = jnp.zeros((16,), jnp.float32)
        return c

    lax.fori_loop(0, ZB // 16, zero, 0)
    pltpu.sync_copy(zbuf.at[pl.ds(0, ROWS_PER_TILE)],
                    acc.at[pl.ds(sid * ROWS_PER_TILE, ROWS_PER_TILE)])
    plsc.subcore_barrier()

    def body(g, c):
        pltpu.sync_copy(dst3.at[wid, pl.ds(g * CH, CH)], dstb)
        for b in range(CH):
            pltpu.sync_copy(ones_v, acc.at[dstb.at[b]], add=True)
        return c

    lax.fori_loop(0, G, body, 0)
    plsc.subcore_barrier()
    off = sid * ROWS_PER_TILE
    pltpu.sync_copy(acc.at[pl.ds(off, ROWS_PER_TILE)],
                    out.at[pl.ds(cid * NP + off, ROWS_PER_TILE)])


@functools.partial(
    pl.kernel,
    mesh=_mesh,
    compiler_params=pltpu.CompilerParams(use_tc_tiling_on_sc=False),
    out_type=jax.ShapeDtypeStruct((NC, NP, HID), jnp.float32),
    scratch_types=[
        pltpu.VMEM_SHARED((NP, HID), jnp.float32),
        pltpu.VMEM((CH, LANES), jnp.int32),
        pltpu.VMEM((CH, LANES), jnp.int32),
        pltpu.VMEM((CH, LANES, HID), jnp.float32),
        pltpu.SemaphoreType.DMA,
    ],
)
def _mp_kernel(hs, src3, dst3, out, acc, srcb, dstb, rows, sem):
    cid = lax.axis_index("c")
    sid = lax.axis_index("s")
    wid = sid * NC + cid
    off = sid * ROWS_PER_TILE
    pltpu.sync_copy(hs.at[pl.ds(off, ROWS_PER_TILE)],
                    acc.at[pl.ds(off, ROWS_PER_TILE)])
    plsc.subcore_barrier()

    def body(g, c):
        pltpu.sync_copy(src3.at[wid, pl.ds(g * CH, CH)], srcb)
        pltpu.sync_copy(dst3.at[wid, pl.ds(g * CH, CH)], dstb)
        handles = [pltpu.async_copy(hs.at[srcb.at[b]], rows.at[b], sem)
                   for b in range(CH)]
        for h in handles:
            h.wait()
        for b in range(CH):
            pltpu.sync_copy(rows.at[b], acc.at[dstb.at[b]], add=True)
        return c

    lax.fori_loop(0, G, body, 0)
    plsc.subcore_barrier()
    pltpu.sync_copy(acc.at[pl.ds(off, ROWS_PER_TILE)],
                    out.at[cid, pl.ds(off, ROWS_PER_TILE)])


BMN = 512
_GRID1 = (N + BMN - 1) // BMN
BR = 784
_GRIDP = (RP + BR - 1) // BR


def _dinvp_body(d0_ref, d1_ref, m_ref, o_ref):
    dinv = lax.rsqrt(d0_ref[...] + d1_ref[...] + 1.0)
    o_ref[...] = jnp.dot(dinv, m_ref[...], preferred_element_type=jnp.float32)


def _dinvp(d0, d1, m):
    return pl.pallas_call(
        _dinvp_body,
        grid=(_GRIDP,),
        in_specs=[
            pl.BlockSpec((BR, 8), lambda i: (i, 0)),
            pl.BlockSpec((BR, 8), lambda i: (i, 0)),
            pl.BlockSpec((8, LANES), lambda i: (0, 0)),
        ],
        out_specs=pl.BlockSpec((BR, LANES), lambda i: (i, 0)),
        out_shape=jax.ShapeDtypeStruct((RP, LANES), jnp.float32),
    )(d0, d1, m)


def _tc1_body(x_ref, w_ref, o_ref):
    o_ref[...] = lax.dot_general(x_ref[...], w_ref[...],
                                 (((0,), (0,)), ((), ())),
                                 preferred_element_type=jnp.float32)


def _tc1(xT, W1):
    return pl.pallas_call(
        _tc1_body,
        grid=(_GRID1,),
        in_specs=[
            pl.BlockSpec((F_IN, BMN), lambda i: (0, i)),
            pl.BlockSpec((F_IN, HID), lambda i: (0, 0)),
        ],
        out_specs=pl.BlockSpec((BMN, HID), lambda i: (i, 0)),
        out_shape=jax.ShapeDtypeStruct((NP, HID), jnp.float32),
    )(xT, W1)


def _mul_body(a_ref, b_ref, o_ref):
    o_ref[...] = a_ref[...] * b_ref[...]


def _mul(a, b):
    return pl.pallas_call(
        _mul_body,
        grid=(_GRIDP,),
        in_specs=[
            pl.BlockSpec((BR, LANES), lambda i: (i, 0)),
            pl.BlockSpec((BR, LANES), lambda i: (i, 0)),
        ],
        out_specs=pl.BlockSpec((BR, LANES), lambda i: (i, 0)),
        out_shape=jax.ShapeDtypeStruct((RP, LANES), jnp.float32),
    )(a, b)


def _mid_body(p_ref, hs_ref, dv_ref, b_ref, w_ref, o_ref):
    dv = dv_ref[...]
    agg = p_ref[0] + p_ref[1] - hs_ref[...]
    a = agg * dv + b_ref[...]
    o1 = jnp.where(a > 0, a, jnp.exp(a) - 1.0)
    o_ref[...] = jnp.dot(o1, w_ref[...],
                         preferred_element_type=jnp.float32) * dv


def _mid(p, hs, dvp, b, Wbd):
    return pl.pallas_call(
        _mid_body,
        grid=(_GRIDP,),
        in_specs=[
            pl.BlockSpec((NC, BR, LANES), lambda i: (0, i, 0)),
            pl.BlockSpec((BR, LANES), lambda i: (i, 0)),
            pl.BlockSpec((BR, LANES), lambda i: (i, 0)),
            pl.BlockSpec((1, LANES), lambda i: (0, 0)),
            pl.BlockSpec((LANES, LANES), lambda i: (0, 0)),
        ],
        out_specs=pl.BlockSpec((BR, LANES), lambda i: (i, 0)),
        out_shape=jax.ShapeDtypeStruct((RP, LANES), jnp.float32),
    )(p, hs, dvp, b, Wbd)


def _fin_body(p_ref, hs_ref, dv_ref, b_ref, s_ref, o_ref):
    dv = dv_ref[...]
    agg = p_ref[0] + p_ref[1] - hs_ref[...]
    a = agg * dv + b_ref[...]
    z = jnp.where(a > 0, a, jnp.exp(a) - 1.0)
    col = lax.broadcasted_iota(jnp.int32, (BR, LANES), 1)
    valid = lax.rem(col, HID) < NCLS
    zm = jnp.where(valid, z, -jnp.inf)
    m = jnp.max(zm, axis=1, keepdims=True)
    e = jnp.where(valid, jnp.exp(z - m), 0.0)
    s = jnp.dot(e, s_ref[...], preferred_element_type=jnp.float32)
    o_ref[...] = z - m - jnp.log(s)


def _fin(p, hs, dvp, b, S):
    return pl.pallas_call(
        _fin_body,
        grid=(_GRIDP,),
        in_specs=[
            pl.BlockSpec((NC, BR, LANES), lambda i: (0, i, 0)),
            pl.BlockSpec((BR, LANES), lambda i: (i, 0)),
            pl.BlockSpec((BR, LANES), lambda i: (i, 0)),
            pl.BlockSpec((1, LANES), lambda i: (0, 0)),
            pl.BlockSpec((LANES, LANES), lambda i: (0, 0)),
        ],
        out_specs=pl.BlockSpec((BR, LANES), lambda i: (i, 0)),
        out_shape=jax.ShapeDtypeStruct((RP, LANES), jnp.float32),
    )(p, hs, dvp, b, S)


def kernel(x, edge_index, W1, b1, W2, b2, W3, b3):
    f32 = jnp.float32
    src = edge_index[0]
    dst = edge_index[1]
    pad = E2 - src.shape[0]
    src3 = jnp.concatenate(
        [src, jnp.zeros((pad,), jnp.int32)]).reshape(NW, T, LANES)
    dump = N + jnp.arange(pad, dtype=jnp.int32) % NDUMP
    dst3 = jnp.concatenate([dst, dump]).reshape(NW, T, LANES)

    degp = _deg_kernel(dst3)
    d0 = degp[:NP].reshape(RP, 8)
    d1 = degp[NP:].reshape(RP, 8)
    eye8 = jnp.eye(8, dtype=f32)
    M = jnp.kron(eye8, jnp.ones((1, HID), f32))
    dvp = _dinvp(d0, d1, M)

    W2bd = jnp.kron(eye8, W2)
    W3bd = jnp.kron(eye8, jnp.pad(W3, ((0, 0), (0, HID - NCLS))))
    S16 = jnp.kron(eye8, jnp.ones((HID, HID), f32))
    b1t = jnp.tile(b1, 8).reshape(1, LANES)
    b2t = jnp.tile(b2, 8).reshape(1, LANES)
    b3t = jnp.tile(jnp.pad(b3, (0, HID - NCLS)), 8).reshape(1, LANES)

    hraw = _tc1(x.T, W1).reshape(RP, LANES)
    hs1 = _mul(hraw, dvp)
    p1 = _mp_kernel(hs1.reshape(NP, HID), src3, dst3)
    hs2 = _mid(p1.reshape(NC, RP, LANES), hs1, dvp, b1t, W2bd)
    p2 = _mp_kernel(hs2.reshape(NP, HID), src3, dst3)
    hs3 = _mid(p2.reshape(NC, RP, LANES), hs2, dvp, b2t, W3bd)
    p3 = _mp_kernel(hs3.reshape(NP, HID), src3, dst3)
    outP = _fin(p3.reshape(NC, RP, LANES), hs3, dvp, b3t, S16)
    return outP.reshape(NP, HID)[:N, :NCLS]


# speedup vs baseline: 1.7353x; 1.1679x over previous
import functools

import jax
import jax.numpy as jnp
from jax import lax
from jax.experimental import pallas as pl
from jax.experimental.pallas import tpu as pltpu
from jax.experimental.pallas import tpu_sc as plsc

N = 50000
F_IN = 1433
HID = 16
NCLS = 7

NC = 2
NS = 16
NW = NC * NS
LANES = 128
CH = 8
T = 392
E2 = NW * T * LANES
G = T // CH

NP = 50048
RP = NP // 8
ROWS_PER_TILE = NP // NS
ZB = 3136
NDUMP = 48

_mesh = plsc.VectorSubcoreMesh(core_axis_name="c", subcore_axis_name="s")


@functools.partial(
    pl.kernel,
    mesh=_mesh,
    compiler_params=pltpu.CompilerParams(use_tc_tiling_on_sc=False),
    out_type=jax.ShapeDtypeStruct((NC * NP,), jnp.float32),
    scratch_types=[
        pltpu.VMEM_SHARED((NP,), jnp.float32),
        pltpu.VMEM((CH, LANES), jnp.int32),
        pltpu.VMEM((LANES,), jnp.float32),
        pltpu.VMEM((ZB,), jnp.float32),
    ],
)
def _deg_kernel(dst3, out, acc, dstb, ones_v, zbuf):
    cid = lax.axis_index("c")
    sid = lax.axis_index("s")
    wid = sid * NC + cid
    for i in range(LANES // 16):
        ones_v[pl.ds(i * 16, 16)] = jnp.ones((16,), jnp.float32)

    def zero(i, c):
        zbuf[pl.ds(i * 16, 16)] = jnp.zeros((16,), jnp.float32)
        return c

    lax.fori_loop(0, ZB // 16, zero, 0)
    pltpu.sync_copy(zbuf.at[pl.ds(0, ROWS_PER_TILE)],
                    acc.at[pl.ds(sid * ROWS_PER_TILE, ROWS_PER_TILE)])
    plsc.subcore_barrier()

    def body(g, c):
        pltpu.sync_copy(dst3.at[wid, pl.ds(g * CH, CH)], dstb)
        for b in range(CH):
            pltpu.sync_copy(ones_v, acc.at[dstb.at[b]], add=True)
        return c

    lax.fori_loop(0, G, body, 0)
    plsc.subcore_barrier()
    off = sid * ROWS_PER_TILE
    pltpu.sync_copy(acc.at[pl.ds(off, ROWS_PER_TILE)],
                    out.at[pl.ds(cid * NP + off, ROWS_PER_TILE)])


@functools.partial(
    pl.kernel,
    mesh=_mesh,
    compiler_params=pltpu.CompilerParams(use_tc_tiling_on_sc=False),
    out_type=jax.ShapeDtypeStruct((NC, NP, HID), jnp.float32),
    scratch_types=[
        pltpu.VMEM_SHARED((NP, HID), jnp.float32),
        pltpu.VMEM((2, CH, LANES), jnp.int32),
        pltpu.VMEM((2, CH, LANES), jnp.int32),
        pltpu.VMEM((2, CH, LANES, HID), jnp.float32),
        pltpu.SemaphoreType.DMA,
        pltpu.SemaphoreType.DMA,
    ],
)
def _mp_kernel(hs, src3, dst3, out, acc, srcb, dstb, rows, sem, sem2):
    cid = lax.axis_index("c")
    sid = lax.axis_index("s")
    wid = sid * NC + cid
    off = sid * ROWS_PER_TILE
    pltpu.sync_copy(hs.at[pl.ds(off, ROWS_PER_TILE)],
                    acc.at[pl.ds(off, ROWS_PER_TILE)])
    plsc.subcore_barrier()

    pltpu.sync_copy(src3.at[wid, pl.ds(0, CH)], srcb.at[0])
    pltpu.sync_copy(dst3.at[wid, pl.ds(0, CH)], dstb.at[0])
    for b in range(CH):
        pltpu.async_copy(hs.at[srcb.at[0, b]], rows.at[0, b], sem)

    def body(g, c):
        p = lax.rem(g, 2)
        q = 1 - p

        for b in range(CH):
            pltpu.make_async_copy(hs.at[pl.ds(0, LANES)],
                                  rows.at[p, b], sem).wait()

        @pl.when(g > 0)
        def _drain_sc():
            for b in range(CH):
                pltpu.make_async_copy(rows.at[q, b],
                                      acc.at[pl.ds(0, LANES)], sem2).wait()

        @pl.when(g < G - 1)
        def _stage_fire():
            pltpu.sync_copy(src3.at[wid, pl.ds((g + 1) * CH, CH)],
                            srcb.at[q])
            pltpu.sync_copy(dst3.at[wid, pl.ds((g + 1) * CH, CH)],
                            dstb.at[q])
            for b in range(CH):
                pltpu.async_copy(hs.at[srcb.at[q, b]], rows.at[q, b], sem)

        for b in range(CH):
            pltpu.async_copy(rows.at[p, b], acc.at[dstb.at[p, b]], sem2,
                             add=True)
        return c

    lax.fori_loop(0, G, body, 0)
    lastp = (G - 1) % 2
    for b in range(CH):
        pltpu.make_async_copy(rows.at[lastp, b],
                              acc.at[pl.ds(0, LANES)], sem2).wait()
    plsc.subcore_barrier()
    pltpu.sync_copy(acc.at[pl.ds(off, ROWS_PER_TILE)],
                    out.at[cid, pl.ds(off, ROWS_PER_TILE)])


BMN = 512
_GRID1 = (N + BMN - 1) // BMN
BR = 784
_GRIDP = (RP + BR - 1) // BR


def _dinvp_body(d0_ref, d1_ref, m_ref, o_ref):
    dinv = lax.rsqrt(d0_ref[...] + d1_ref[...] + 1.0)
    o_ref[...] = jnp.dot(dinv, m_ref[...], preferred_element_type=jnp.float32)


def _dinvp(d0, d1, m):
    return pl.pallas_call(
        _dinvp_body,
        grid=(_GRIDP,),
        in_specs=[
            pl.BlockSpec((BR, 8), lambda i: (i, 0)),
            pl.BlockSpec((BR, 8), lambda i: (i, 0)),
            pl.BlockSpec((8, LANES), lambda i: (0, 0)),
        ],
        out_specs=pl.BlockSpec((BR, LANES), lambda i: (i, 0)),
        out_shape=jax.ShapeDtypeStruct((RP, LANES), jnp.float32),
    )(d0, d1, m)


def _tc1_body(x_ref, w_ref, o_ref):
    o_ref[...] = lax.dot_general(x_ref[...], w_ref[...],
                                 (((0,), (0,)), ((), ())),
                                 preferred_element_type=jnp.float32)


def _tc1(xT, W1):
    return pl.pallas_call(
        _tc1_body,
        grid=(_GRID1,),
        in_specs=[
            pl.BlockSpec((F_IN, BMN), lambda i: (0, i)),
            pl.BlockSpec((F_IN, HID), lambda i: (0, 0)),
        ],
        out_specs=pl.BlockSpec((BMN, HID), lambda i: (i, 0)),
        out_shape=jax.ShapeDtypeStruct((NP, HID), jnp.float32),
    )(xT, W1)


def _mul_body(a_ref, b_ref, o_ref):
    o_ref[...] = a_ref[...] * b_ref[...]


def _mul(a, b):
    return pl.pallas_call(
        _mul_body,
        grid=(_GRIDP,),
        in_specs=[
            pl.BlockSpec((BR, LANES), lambda i: (i, 0)),
            pl.BlockSpec((BR, LANES), lambda i: (i, 0)),
        ],
        out_specs=pl.BlockSpec((BR, LANES), lambda i: (i, 0)),
        out_shape=jax.ShapeDtypeStruct((RP, LANES), jnp.float32),
    )(a, b)


def _mid_body(p_ref, hs_ref, dv_ref, b_ref, w_ref, o_ref):
    dv = dv_ref[...]
    agg = p_ref[0] + p_ref[1] - hs_ref[...]
    a = agg * dv + b_ref[...]
    o1 = jnp.where(a > 0, a, jnp.exp(a) - 1.0)
    o_ref[...] = jnp.dot(o1, w_ref[...],
                         preferred_element_type=jnp.float32) * dv


def _mid(p, hs, dvp, b, Wbd):
    return pl.pallas_call(
        _mid_body,
        grid=(_GRIDP,),
        in_specs=[
            pl.BlockSpec((NC, BR, LANES), lambda i: (0, i, 0)),
            pl.BlockSpec((BR, LANES), lambda i: (i, 0)),
            pl.BlockSpec((BR, LANES), lambda i: (i, 0)),
            pl.BlockSpec((1, LANES), lambda i: (0, 0)),
            pl.BlockSpec((LANES, LANES), lambda i: (0, 0)),
        ],
        out_specs=pl.BlockSpec((BR, LANES), lambda i: (i, 0)),
        out_shape=jax.ShapeDtypeStruct((RP, LANES), jnp.float32),
    )(p, hs, dvp, b, Wbd)


def _fin_body(p_ref, hs_ref, dv_ref, b_ref, s_ref, o_ref):
    dv = dv_ref[...]
    agg = p_ref[0] + p_ref[1] - hs_ref[...]
    a = agg * dv + b_ref[...]
    z = jnp.where(a > 0, a, jnp.exp(a) - 1.0)
    col = lax.broadcasted_iota(jnp.int32, (BR, LANES), 1)
    valid = lax.rem(col, HID) < NCLS
    zm = jnp.where(valid, z, -jnp.inf)
    m = jnp.max(zm, axis=1, keepdims=True)
    e = jnp.where(valid, jnp.exp(z - m), 0.0)
    s = jnp.dot(e, s_ref[...], preferred_element_type=jnp.float32)
    o_ref[...] = z - m - jnp.log(s)


def _fin(p, hs, dvp, b, S):
    return pl.pallas_call(
        _fin_body,
        grid=(_GRIDP,),
        in_specs=[
            pl.BlockSpec((NC, BR, LANES), lambda i: (0, i, 0)),
            pl.BlockSpec((BR, LANES), lambda i: (i, 0)),
            pl.BlockSpec((BR, LANES), lambda i: (i, 0)),
            pl.BlockSpec((1, LANES), lambda i: (0, 0)),
            pl.BlockSpec((LANES, LANES), lambda i: (0, 0)),
        ],
        out_specs=pl.BlockSpec((BR, LANES), lambda i: (i, 0)),
        out_shape=jax.ShapeDtypeStruct((RP, LANES), jnp.float32),
    )(p, hs, dvp, b, S)


def kernel(x, edge_index, W1, b1, W2, b2, W3, b3):
    f32 = jnp.float32
    src = edge_index[0]
    dst = edge_index[1]
    pad = E2 - src.shape[0]
    src3 = jnp.concatenate(
        [src, jnp.zeros((pad,), jnp.int32)]).reshape(NW, T, LANES)
    dump = N + jnp.arange(pad, dtype=jnp.int32) % NDUMP
    dst3 = jnp.concatenate([dst, dump]).reshape(NW, T, LANES)

    degp = _deg_kernel(dst3)
    d0 = degp[:NP].reshape(RP, 8)
    d1 = degp[NP:].reshape(RP, 8)
    eye8 = jnp.eye(8, dtype=f32)
    M = jnp.kron(eye8, jnp.ones((1, HID), f32))
    dvp = _dinvp(d0, d1, M)

    W2bd = jnp.kron(eye8, W2)
    W3bd = jnp.kron(eye8, jnp.pad(W3, ((0, 0), (0, HID - NCLS))))
    S16 = jnp.kron(eye8, jnp.ones((HID, HID), f32))
    b1t = jnp.tile(b1, 8).reshape(1, LANES)
    b2t = jnp.tile(b2, 8).reshape(1, LANES)
    b3t = jnp.tile(jnp.pad(b3, (0, HID - NCLS)), 8).reshape(1, LANES)

    hraw = _tc1(x.T, W1).reshape(RP, LANES)
    hs1 = _mul(hraw, dvp)
    p1 = _mp_kernel(hs1.reshape(NP, HID), src3, dst3)
    hs2 = _mid(p1.reshape(NC, RP, LANES), hs1, dvp, b1t, W2bd)
    p2 = _mp_kernel(hs2.reshape(NP, HID), src3, dst3)
    hs3 = _mid(p2.reshape(NC, RP, LANES), hs2, dvp, b2t, W3bd)
    p3 = _mp_kernel(hs3.reshape(NP, HID), src3, dst3)
    outP = _fin(p3.reshape(NC, RP, LANES), hs3, dvp, b3t, S16)
    return outP.reshape(NP, HID)[:N, :NCLS]


# speedup vs baseline: 1.9241x; 1.1088x over previous
import functools

import jax
import jax.numpy as jnp
from jax import lax
from jax.experimental import pallas as pl
from jax.experimental.pallas import tpu as pltpu
from jax.experimental.pallas import tpu_sc as plsc

N = 50000
F_IN = 1433
HID = 16
NCLS = 7

NC = 2
NS = 16
NW = NC * NS
LANES = 128
CH = 14
T = 392
E2 = NW * T * LANES
G = T // CH

NP = 50048
RP = NP // 8
ROWS_PER_TILE = NP // NS
ZB = 3136
NDUMP = 48

_mesh = plsc.VectorSubcoreMesh(core_axis_name="c", subcore_axis_name="s")


@functools.partial(
    pl.kernel,
    mesh=_mesh,
    compiler_params=pltpu.CompilerParams(use_tc_tiling_on_sc=False),
    out_type=jax.ShapeDtypeStruct((NC * NP,), jnp.float32),
    scratch_types=[
        pltpu.VMEM_SHARED((NP,), jnp.float32),
        pltpu.VMEM((CH, LANES), jnp.int32),
        pltpu.VMEM((LANES,), jnp.float32),
        pltpu.VMEM((ZB,), jnp.float32),
    ],
)
def _deg_kernel(dst3, out, acc, dstb, ones_v, zbuf):
    cid = lax.axis_index("c")
    sid = lax.axis_index("s")
    wid = sid * NC + cid
    for i in range(LANES // 16):
        ones_v[pl.ds(i * 16, 16)] = jnp.ones((16,), jnp.float32)

    def zero(i, c):
        zbuf[pl.ds(i * 16, 16)] = jnp.zeros((16,), jnp.float32)
        return c

    lax.fori_loop(0, ZB // 16, zero, 0)
    pltpu.sync_copy(zbuf.at[pl.ds(0, ROWS_PER_TILE)],
                    acc.at[pl.ds(sid * ROWS_PER_TILE, ROWS_PER_TILE)])
    plsc.subcore_barrier()

    def body(g, c):
        pltpu.sync_copy(dst3.at[wid, pl.ds(g * CH, CH)], dstb)
        for b in range(CH):
            pltpu.sync_copy(ones_v, acc.at[dstb.at[b]], add=True)
        return c

    lax.fori_loop(0, G, body, 0)
    plsc.subcore_barrier()
    off = sid * ROWS_PER_TILE
    pltpu.sync_copy(acc.at[pl.ds(off, ROWS_PER_TILE)],
                    out.at[pl.ds(cid * NP + off, ROWS_PER_TILE)])


@functools.partial(
    pl.kernel,
    mesh=_mesh,
    compiler_params=pltpu.CompilerParams(use_tc_tiling_on_sc=False),
    out_type=jax.ShapeDtypeStruct((NC, NP, HID), jnp.float32),
    scratch_types=[
        pltpu.VMEM_SHARED((NP, HID), jnp.float32),
        pltpu.VMEM((2, CH, LANES), jnp.int32),
        pltpu.VMEM((2, CH, LANES), jnp.int32),
        pltpu.VMEM((2, CH, LANES, HID), jnp.float32),
        pltpu.SemaphoreType.DMA,
        pltpu.SemaphoreType.DMA,
    ],
)
def _mp_kernel(hs, src3, dst3, out, acc, srcb, dstb, rows, sem, sem2):
    cid = lax.axis_index("c")
    sid = lax.axis_index("s")
    wid = sid * NC + cid
    off = sid * ROWS_PER_TILE
    pltpu.sync_copy(hs.at[pl.ds(off, ROWS_PER_TILE)],
                    acc.at[pl.ds(off, ROWS_PER_TILE)])
    plsc.subcore_barrier()

    pltpu.sync_copy(src3.at[wid, pl.ds(0, CH)], srcb.at[0])
    pltpu.sync_copy(dst3.at[wid, pl.ds(0, CH)], dstb.at[0])
    for b in range(CH):
        pltpu.async_copy(hs.at[srcb.at[0, b]], rows.at[0, b], sem)

    def body(g, c):
        p = lax.rem(g, 2)
        q = 1 - p

        for b in range(CH):
            pltpu.make_async_copy(hs.at[pl.ds(0, LANES)],
                                  rows.at[p, b], sem).wait()

        @pl.when(g > 0)
        def _drain_sc():
            for b in range(CH):
                pltpu.make_async_copy(rows.at[q, b],
                                      acc.at[pl.ds(0, LANES)], sem2).wait()

        @pl.when(g < G - 1)
        def _stage_fire():
            pltpu.sync_copy(src3.at[wid, pl.ds((g + 1) * CH, CH)],
                            srcb.at[q])
            pltpu.sync_copy(dst3.at[wid, pl.ds((g + 1) * CH, CH)],
                            dstb.at[q])
            for b in range(CH):
                pltpu.async_copy(hs.at[srcb.at[q, b]], rows.at[q, b], sem)

        for b in range(CH):
            pltpu.async_copy(rows.at[p, b], acc.at[dstb.at[p, b]], sem2,
                             add=True)
        return c

    lax.fori_loop(0, G, body, 0)
    lastp = (G - 1) % 2
    for b in range(CH):
        pltpu.make_async_copy(rows.at[lastp, b],
                              acc.at[pl.ds(0, LANES)], sem2).wait()
    plsc.subcore_barrier()
    pltpu.sync_copy(acc.at[pl.ds(off, ROWS_PER_TILE)],
                    out.at[cid, pl.ds(off, ROWS_PER_TILE)])


BMN = 512
_GRID1 = (N + BMN - 1) // BMN
BR = 784
_GRIDP = (RP + BR - 1) // BR


def _dinvp_body(d0_ref, d1_ref, m_ref, o_ref):
    dinv = lax.rsqrt(d0_ref[...] + d1_ref[...] + 1.0)
    o_ref[...] = jnp.dot(dinv, m_ref[...], preferred_element_type=jnp.float32)


def _dinvp(d0, d1, m):
    return pl.pallas_call(
        _dinvp_body,
        grid=(_GRIDP,),
        in_specs=[
            pl.BlockSpec((BR, 8), lambda i: (i, 0)),
            pl.BlockSpec((BR, 8), lambda i: (i, 0)),
            pl.BlockSpec((8, LANES), lambda i: (0, 0)),
        ],
        out_specs=pl.BlockSpec((BR, LANES), lambda i: (i, 0)),
        out_shape=jax.ShapeDtypeStruct((RP, LANES), jnp.float32),
    )(d0, d1, m)


def _tc1_body(x_ref, w_ref, o_ref):
    o_ref[...] = lax.dot_general(x_ref[...], w_ref[...],
                                 (((0,), (0,)), ((), ())),
                                 preferred_element_type=jnp.float32)


def _tc1(xT, W1):
    return pl.pallas_call(
        _tc1_body,
        grid=(_GRID1,),
        in_specs=[
            pl.BlockSpec((F_IN, BMN), lambda i: (0, i)),
            pl.BlockSpec((F_IN, HID), lambda i: (0, 0)),
        ],
        out_specs=pl.BlockSpec((BMN, HID), lambda i: (i, 0)),
        out_shape=jax.ShapeDtypeStruct((NP, HID), jnp.float32),
    )(xT, W1)


def _mul_body(a_ref, b_ref, o_ref):
    o_ref[...] = a_ref[...] * b_ref[...]


def _mul(a, b):
    return pl.pallas_call(
        _mul_body,
        grid=(_GRIDP,),
        in_specs=[
            pl.BlockSpec((BR, LANES), lambda i: (i, 0)),
            pl.BlockSpec((BR, LANES), lambda i: (i, 0)),
        ],
        out_specs=pl.BlockSpec((BR, LANES), lambda i: (i, 0)),
        out_shape=jax.ShapeDtypeStruct((RP, LANES), jnp.float32),
    )(a, b)


def _mid_body(p_ref, hs_ref, dv_ref, b_ref, w_ref, o_ref):
    dv = dv_ref[...]
    agg = p_ref[0] + p_ref[1] - hs_ref[...]
    a = agg * dv + b_ref[...]
    o1 = jnp.where(a > 0, a, jnp.exp(a) - 1.0)
    o_ref[...] = jnp.dot(o1, w_ref[...],
                         preferred_element_type=jnp.float32) * dv


def _mid(p, hs, dvp, b, Wbd):
    return pl.pallas_call(
        _mid_body,
        grid=(_GRIDP,),
        in_specs=[
            pl.BlockSpec((NC, BR, LANES), lambda i: (0, i, 0)),
            pl.BlockSpec((BR, LANES), lambda i: (i, 0)),
            pl.BlockSpec((BR, LANES), lambda i: (i, 0)),
            pl.BlockSpec((1, LANES), lambda i: (0, 0)),
            pl.BlockSpec((LANES, LANES), lambda i: (0, 0)),
        ],
        out_specs=pl.BlockSpec((BR, LANES), lambda i: (i, 0)),
        out_shape=jax.ShapeDtypeStruct((RP, LANES), jnp.float32),
    )(p, hs, dvp, b, Wbd)


def _fin_body(p_ref, hs_ref, dv_ref, b_ref, s_ref, o_ref):
    dv = dv_ref[...]
    agg = p_ref[0] + p_ref[1] - hs_ref[...]
    a = agg * dv + b_ref[...]
    z = jnp.where(a > 0, a, jnp.exp(a) - 1.0)
    col = lax.broadcasted_iota(jnp.int32, (BR, LANES), 1)
    valid = lax.rem(col, HID) < NCLS
    zm = jnp.where(valid, z, -jnp.inf)
    m = jnp.max(zm, axis=1, keepdims=True)
    e = jnp.where(valid, jnp.exp(z - m), 0.0)
    s = jnp.dot(e, s_ref[...], preferred_element_type=jnp.float32)
    o_ref[...] = z - m - jnp.log(s)


def _fin(p, hs, dvp, b, S):
    return pl.pallas_call(
        _fin_body,
        grid=(_GRIDP,),
        in_specs=[
            pl.BlockSpec((NC, BR, LANES), lambda i: (0, i, 0)),
            pl.BlockSpec((BR, LANES), lambda i: (i, 0)),
            pl.BlockSpec((BR, LANES), lambda i: (i, 0)),
            pl.BlockSpec((1, LANES), lambda i: (0, 0)),
            pl.BlockSpec((LANES, LANES), lambda i: (0, 0)),
        ],
        out_specs=pl.BlockSpec((BR, LANES), lambda i: (i, 0)),
        out_shape=jax.ShapeDtypeStruct((RP, LANES), jnp.float32),
    )(p, hs, dvp, b, S)


def kernel(x, edge_index, W1, b1, W2, b2, W3, b3):
    f32 = jnp.float32
    src = edge_index[0]
    dst = edge_index[1]
    pad = E2 - src.shape[0]
    src3 = jnp.concatenate(
        [src, jnp.zeros((pad,), jnp.int32)]).reshape(NW, T, LANES)
    dump = N + jnp.arange(pad, dtype=jnp.int32) % NDUMP
    dst3 = jnp.concatenate([dst, dump]).reshape(NW, T, LANES)

    degp = _deg_kernel(dst3)
    d0 = degp[:NP].reshape(RP, 8)
    d1 = degp[NP:].reshape(RP, 8)
    eye8 = jnp.eye(8, dtype=f32)
    M = jnp.kron(eye8, jnp.ones((1, HID), f32))
    dvp = _dinvp(d0, d1, M)

    W2bd = jnp.kron(eye8, W2)
    W3bd = jnp.kron(eye8, jnp.pad(W3, ((0, 0), (0, HID - NCLS))))
    S16 = jnp.kron(eye8, jnp.ones((HID, HID), f32))
    b1t = jnp.tile(b1, 8).reshape(1, LANES)
    b2t = jnp.tile(b2, 8).reshape(1, LANES)
    b3t = jnp.tile(jnp.pad(b3, (0, HID - NCLS)), 8).reshape(1, LANES)

    hraw = _tc1(x.T, W1).reshape(RP, LANES)
    hs1 = _mul(hraw, dvp)
    p1 = _mp_kernel(hs1.reshape(NP, HID), src3, dst3)
    hs2 = _mid(p1.reshape(NC, RP, LANES), hs1, dvp, b1t, W2bd)
    p2 = _mp_kernel(hs2.reshape(NP, HID), src3, dst3)
    hs3 = _mid(p2.reshape(NC, RP, LANES), hs2, dvp, b2t, W3bd)
    p3 = _mp_kernel(hs3.reshape(NP, HID), src3, dst3)
    outP = _fin(p3.reshape(NC, RP, LANES), hs3, dvp, b3t, S16)
    return outP.reshape(NP, HID)[:N, :NCLS]
